# baseline (speedup 1.0000x reference)
import functools

import jax
import jax.numpy as jnp
from jax import lax
from jax.experimental import pallas as pl
from jax.experimental.pallas import tpu as pltpu
from jax.experimental.pallas import tpu_sc as plsc

_BLK = 128
_RB = 32
_NZ = 32
_NTILES = 16


def _build_step(n, d, realh, hp, np_):
    rpt = hp // _NTILES
    kblocks = rpt // _RB
    f32 = jnp.float32
    i32 = jnp.int32
    mesh = plsc.VectorSubcoreMesh(core_axis_name="c", subcore_axis_name="s")

    @functools.partial(
        pl.kernel,
        out_type=(
            jax.ShapeDtypeStruct((np_, d), f32),
            jax.ShapeDtypeStruct((np_ + _NZ, d), f32),
            jax.ShapeDtypeStruct((2, d), f32),
            jax.ShapeDtypeStruct((2, d), f32),
        ),
        mesh=mesh,
        scratch_types=[
            pltpu.VMEM_SHARED((hp, d), f32),
            pltpu.VMEM_SHARED((_NTILES, d), f32),
            pltpu.VMEM((2, 32, _BLK), i32),
            pltpu.VMEM((2, 32, _BLK), i32),
            pltpu.VMEM((2, _BLK, d), f32),
            pltpu.VMEM((_RB, d), f32),
            pltpu.VMEM((_RB, d), f32),
            pltpu.VMEM((_RB, d), f32),
            pltpu.VMEM((_RB, d), f32),
            pltpu.VMEM((2, d), f32),
            pltpu.VMEM((4, d), f32),
            pltpu.VMEM((_NTILES, d), f32),
            pltpu.VMEM((rpt + 16,), f32),
            pltpu.VMEM((rpt + 16,), f32),
            pltpu.VMEM((16,), i32),
            pltpu.SemaphoreType.REGULAR,
            pltpu.SemaphoreType.DMA((2,)),
            pltpu.SemaphoreType.DMA((2,)),
        ],
    )
    def step(sa_hbm, cs_hbm, cons_hbm, kb_hbm, dis_hbm, g_hbm, colg_hbm,
             rowl_hbm, sz_hbm,
             outp_hbm, sb_hbm, csa_hbm, csb_hbm,
             agg_sh, part_sh, cg_vm, rl_vm, rows, ablk, kblk, oblk,
             sblk, csb, cb, pbuf, dis_vm, g_vm, sz_vm, bsem,
             gsem, ssem):
        zbuf = oblk
        cid = lax.axis_index("c")
        sid = lax.axis_index("s")
        other = 1 - cid

        pltpu.sync_copy(sz_hbm, sz_vm)
        pltpu.sync_copy(cons_hbm, cb)
        gbase = cid * hp + sid * rpt
        pltpu.sync_copy(dis_hbm.at[pl.ds(gbase, rpt)], dis_vm.at[pl.ds(0, rpt)])
        pltpu.sync_copy(g_hbm.at[pl.ds(gbase, rpt)], g_vm.at[pl.ds(0, rpt)])
        szv = sz_vm[pl.ds(0, 16)]
        b0 = szv[0]
        b1 = szv[1]
        nk = szv[2]
        nblk = jnp.where(cid == 0, b0, b1)
        baseb = cid * b0
        bpt = ((nblk + (_NTILES * 32 - 1)) >> 9) << 5
        jlo = sid * bpt
        nb = jnp.clip(nblk - jlo, 0, bpt)
        tb0 = baseb + jlo
        ngrp = nb >> 1
        inv_n = 1.0 / float(n)
        alv = [cb[0, pl.ds(c * 16, 16)] for c in range(8)]
        amv = [cb[1, pl.ds(c * 16, 16)] for c in range(8)]
        bm1v = [cb[2, pl.ds(c * 16, 16)] for c in range(8)]
        mnv = [cb[3, pl.ds(c * 16, 16)] for c in range(8)]

        def _one_iter(s_in, s_out, cs_in, cs_out, it, do_sync):
            pltpu.sync_copy(cs_in, csb)

            def _zr(r, _):
                for c in range(8):
                    zbuf[r, pl.ds(c * 16, 16)] = jnp.zeros((16,), f32)
                return 0
            lax.fori_loop(0, _RB, _zr, 0)
            for k in range(kblocks):
                pltpu.sync_copy(zbuf,
                                agg_sh.at[pl.ds(sid * rpt + k * _RB, _RB)])
            plsc.subcore_barrier()

            def _stage(j):
                par = (j >> 5) & 1
                boff = pl.multiple_of(tb0 + j, 32)
                pltpu.sync_copy(colg_hbm.at[pl.ds(boff, 32)], cg_vm.at[par])
                pltpu.sync_copy(rowl_hbm.at[pl.ds(boff, 32)], rl_vm.at[par])

            def _gather(j, k):
                par = (j >> 5) & 1
                c = j & 31
                return pltpu.async_copy(s_in.at[cg_vm.at[par].at[c]],
                                        rows.at[k], gsem.at[k])

            def _gwait(j, k):
                par = (j >> 5) & 1
                c = j & 31
                pltpu.make_async_copy(s_in.at[cg_vm.at[par].at[c]],
                                      rows.at[k], gsem.at[k]).wait()

            def _scatter(j, k):
                par = (j >> 5) & 1
                c = j & 31
                return pltpu.async_copy(rows.at[k],
                                        agg_sh.at[rl_vm.at[par].at[c]],
                                        ssem.at[k], add=True)

            def _swait(j, k):
                par = (j >> 5) & 1
                c = j & 31
                pltpu.make_async_copy(rows.at[k],
                                      agg_sh.at[rl_vm.at[par].at[c]],
                                      ssem.at[k]).wait()

            @pl.when(ngrp > 0)
            def _():
                _stage(0)
                for k in range(2):
                    _gather(k, k)

            def _grp(grp, _):
                j0 = 2 * grp
                for k in range(2):
                    _gwait(j0 + k, k)
                    _scatter(j0 + k, k)
                jn0 = j0 + 2

                @pl.when(jn0 < nb)
                def _():
                    @pl.when((jn0 & 31) == 0)
                    def _():
                        _stage(jn0)
                    for k in range(2):
                        _swait(j0 + k, k)
                        _gather(jn0 + k, k)

                @pl.when(jn0 >= nb)
                def _():
                    for k in range(2):
                        _swait(j0 + k, k)
                return 0

            lax.fori_loop(0, ngrp, _grp, 0)
            plsc.subcore_barrier()

            amuv = []
            for c in range(8):
                sl = pl.ds(c * 16, 16)
                mu = (csb[0, sl] + csb[1, sl]) * inv_n
                amuv.append(amv[c] * mu)

            def _bblk(k, accs):
                rb0 = sid * rpt + k * _RB
                grow0 = cid * hp + rb0
                pltpu.sync_copy(agg_sh.at[pl.ds(rb0, _RB)], ablk)
                pltpu.sync_copy(kb_hbm.at[pl.ds(grow0, _RB)], kblk)

                def _rb(r, accs):
                    lrow = rb0 + r
                    dis_r = dis_vm[pl.ds(k * _RB + r, 16)][0]
                    g_r = g_vm[pl.ds(k * _RB + r, 16)][0]
                    val = jnp.where(lrow < realh, 1.0, 0.0).astype(f32)
                    new = []
                    for c in range(8):
                        sl = pl.ds(c * 16, 16)
                        a = ablk[r, sl]
                        v = a * dis_r * alv[c] + amuv[c]
                        kv = kblk[r, sl]
                        bb = bm1v[c] * g_r + 1.0
                        outv = (v * bb + kv * g_r) * val
                        oblk[r, sl] = outv + mnv[c]
                        sblk[r, sl] = outv * dis_r
                        new.append(accs[c] + outv)
                    return tuple(new)

                accs = lax.fori_loop(0, _RB, _rb, accs)
                pltpu.sync_copy(oblk, outp_hbm.at[pl.ds(grow0, _RB)])
                pltpu.sync_copy(sblk, s_out.at[pl.ds(grow0, _RB)])
                return accs

            accs = tuple(jnp.zeros((16,), f32) for _ in range(8))
            accs = lax.fori_loop(0, kblocks, _bblk, accs)
            for c in range(8):
                ablk[0, pl.ds(c * 16, 16)] = accs[c]
            pltpu.sync_copy(ablk.at[pl.ds(0, 1)], part_sh.at[pl.ds(sid, 1)])
            plsc.subcore_barrier()

            @pl.when(sid == 0)
            def _():
                pltpu.sync_copy(part_sh, pbuf)
                for c in range(8):
                    sl = pl.ds(c * 16, 16)
                    tot = pbuf[0, sl]
                    for t in range(1, _NTILES):
                        tot = tot + pbuf[t, sl]
                    ablk[1, sl] = tot
                pltpu.sync_copy(ablk.at[pl.ds(1, 1)],
                                cs_out.at[pl.ds(cid, 1)])

                def _rz(r, _):
                    for c in range(8):
                        zbuf[r, pl.ds(c * 16, 16)] = jnp.zeros((16,), f32)
                    return 0
                lax.fori_loop(0, 16, _rz, 0)
                pltpu.sync_copy(zbuf.at[pl.ds(0, 16)],
                                s_out.at[pl.ds(np_ + cid * 16, 16)])
            plsc.subcore_barrier()
            if do_sync:
                pltpu.core_barrier(bsem, core_axis_name="c")

        def _pair(ph, _):
            it = 2 * ph
            _one_iter(sa_hbm, sb_hbm, cs_hbm, csb_hbm, it, True)
            _one_iter(sb_hbm, sa_hbm, csb_hbm, csa_hbm, it + 1, True)
            return 0

        lax.fori_loop(0, nk >> 1, _pair, 0)

        @pl.when((nk & 1) == 1)
        def _():
            _one_iter(sa_hbm, sb_hbm, cs_hbm, csb_hbm, nk - 1, False)

    return step


def _setup(x, edge_index, know_mask, eta, theta):
    f32 = jnp.float32
    n, d = x.shape
    e = edge_index.shape[1]
    realh = n // 2
    hp = -(-realh // (_NTILES * _RB)) * (_NTILES * _RB)
    np_ = 2 * hp

    row = edge_index[0]
    col = edge_index[1]
    ew = (row != col)
    deg = jax.ops.segment_sum(ew.astype(f32), row, num_segments=n)
    dis = jnp.where(deg > 0, lax.rsqrt(jnp.where(deg > 0, deg, 1.0)), 0.0)

    alpha = (n - 1.0) / (theta * n + n - 1.0)
    beta = 1.0 / alpha / (1.0 / alpha + eta)

    nkm = know_mask.shape[0]
    xk = x[know_mask]
    mean = xk.mean(axis=0)
    upd = jnp.concatenate(
        [xk - mean, jnp.ones((nkm, 1), f32), jnp.zeros((nkm, 15), f32)], 1)
    kext = jnp.zeros((n, d + 16), f32).at[know_mask].set(upd)
    kdense = kext[:, :d]
    gflag = kext[:, d]
    kb = kdense * (1.0 - beta)
    out0 = kdense - mean * (1.0 - gflag)[:, None]

    zpad = hp - realh

    def padrows(a):
        top, bot = a[:realh], a[realh:]
        z = jnp.zeros((zpad,) + a.shape[1:], a.dtype)
        return jnp.concatenate([top, z, bot, z], 0)

    kb_p = padrows(kb)
    dis_p = padrows(dis)
    g_p = padrows(gflag)
    out0_p = padrows(out0)
    s0 = jnp.concatenate([dis_p[:, None] * out0_p, jnp.zeros((_NZ, d), f32)], 0)
    cs0 = jnp.stack([out0[:realh].sum(0), out0[realh:].sum(0)])
    cons = jnp.stack([alpha, 1.0 - alpha, beta - 1.0, mean], 0)

    i32 = jnp.int32
    f = (row >= realh).astype(i32)
    c0cum = jnp.cumsum(1 - f)
    c1cum = jnp.cumsum(f)
    es = c0cum[-1]
    pad0 = (-es) % (32 * _BLK)
    p0 = es + pad0
    cnt1 = e - es
    pblk = -(-(e + 64 * _BLK) // _BLK) + 32
    ptot = pblk * _BLK
    pos = jnp.where(f == 0, c0cum - 1, p0 + c1cum - 1).astype(i32)
    colm = col + zpad * (col >= realh).astype(i32)
    cg_e = jnp.where(row != col, colm, np_ + (pos % _NZ)).astype(i32)
    rl_e = (row - realh * f).astype(i32)
    t = jnp.arange(ptot, dtype=i32)
    colg = (np_ + (t % _NZ)).astype(i32).at[pos].set(cg_e).reshape(pblk, _BLK)
    rowl = jnp.full((ptot,), realh, i32).at[pos].set(rl_e).reshape(pblk, _BLK)
    b0 = p0 // _BLK
    b1 = 32 * (-(-cnt1 // (32 * _BLK)))

    return dict(n=n, d=d, realh=realh, hp=hp, np_=np_,
                s0=s0, out0_p=out0_p, cs0=cs0, cons=cons, kb_p=kb_p,
                dis_p=dis_p, g_p=g_p, colg=colg, rowl=rowl, b0=b0, b1=b1,
                mean=mean)


def kernel(x, y, edge_index, know_mask, eta, theta, num_iter):
    del y
    st = _setup(x, edge_index, know_mask, eta, theta)
    n, d, realh, hp = st["n"], st["d"], st["realh"], st["hp"]
    step = _build_step(n, d, realh, hp, st["np_"])
    ni = jnp.asarray(num_iter, jnp.int32)
    sizes = (jnp.zeros((16,), jnp.int32)
             .at[0].set(st["b0"]).at[1].set(st["b1"]).at[2].set(ni))
    outp, _, _, _ = step(st["s0"], st["cs0"], st["cons"], st["kb_p"],
                            st["dis_p"], st["g_p"], st["colg"], st["rowl"],
                            sizes)
    res = jnp.concatenate([outp[:realh], outp[hp:hp + realh]], 0)
    init = st["out0_p"] + st["mean"]
    init = jnp.concatenate([init[:realh], init[hp:hp + realh]], 0)
    return jnp.where(ni > 0, res, init)


# speedup vs baseline: 1.2575x; 1.2575x over previous
import functools

import jax
import jax.numpy as jnp
from jax import lax
from jax.experimental import pallas as pl
from jax.experimental.pallas import tpu as pltpu
from jax.experimental.pallas import tpu_sc as plsc

_BLK = 128
_RB = 32
_NZ = 32
_NTILES = 16


def _build_step(n, d, realh, hp, np_):
    rpt = hp // _NTILES
    kblocks = rpt // _RB
    f32 = jnp.float32
    i32 = jnp.int32
    mesh = plsc.VectorSubcoreMesh(core_axis_name="c", subcore_axis_name="s")

    @functools.partial(
        pl.kernel,
        out_type=(
            jax.ShapeDtypeStruct((np_, d), f32),
            jax.ShapeDtypeStruct((np_ + _NZ, d), f32),
            jax.ShapeDtypeStruct((2, d), f32),
            jax.ShapeDtypeStruct((2, d), f32),
        ),
        mesh=mesh,
        scratch_types=[
            pltpu.VMEM_SHARED((hp, d), f32),
            pltpu.VMEM_SHARED((_NTILES, d), f32),
            pltpu.VMEM((2, 32, _BLK), i32),
            pltpu.VMEM((2, 32, _BLK), i32),
            pltpu.VMEM((2, _BLK, d), f32),
            pltpu.VMEM((_RB, d), f32),
            pltpu.VMEM((_RB, d), f32),
            pltpu.VMEM((_RB, d), f32),
            pltpu.VMEM((_RB, d), f32),
            pltpu.VMEM((2, d), f32),
            pltpu.VMEM((4, d), f32),
            pltpu.VMEM((_NTILES, d), f32),
            pltpu.VMEM((rpt + 16,), f32),
            pltpu.VMEM((rpt + 16,), f32),
            pltpu.VMEM((16,), i32),
            pltpu.SemaphoreType.REGULAR,
            pltpu.SemaphoreType.DMA((2,)),
            pltpu.SemaphoreType.DMA((2,)),
        ],
    )
    def step(sa_hbm, cs_hbm, cons_hbm, kb_hbm, dis_hbm, g_hbm, colg_hbm,
             rowl_hbm, sz_hbm,
             outp_hbm, sb_hbm, csa_hbm, csb_hbm,
             agg_sh, part_sh, cg_vm, rl_vm, rows, ablk, kblk, oblk,
             sblk, csb, cb, pbuf, dis_vm, g_vm, sz_vm, bsem,
             gsem, ssem):
        zbuf = oblk
        cid = lax.axis_index("c")
        sid = lax.axis_index("s")
        other = 1 - cid

        pltpu.sync_copy(sz_hbm, sz_vm)
        pltpu.sync_copy(cons_hbm, cb)
        gbase = cid * hp + sid * rpt
        pltpu.sync_copy(dis_hbm.at[pl.ds(gbase, rpt)], dis_vm.at[pl.ds(0, rpt)])
        pltpu.sync_copy(g_hbm.at[pl.ds(gbase, rpt)], g_vm.at[pl.ds(0, rpt)])
        szv = sz_vm[pl.ds(0, 16)]
        b0 = szv[0]
        b1 = szv[1]
        nk = szv[2]
        nblk = jnp.where(cid == 0, b0, b1)
        baseb = cid * b0
        bpt = ((nblk + (_NTILES * 32 - 1)) >> 9) << 5
        jlo = sid * bpt
        nb = jnp.clip(nblk - jlo, 0, bpt)
        tb0 = baseb + jlo
        ngrp = nb >> 1
        inv_n = 1.0 / float(n)
        alv = [cb[0, pl.ds(c * 16, 16)] for c in range(8)]
        amv = [cb[1, pl.ds(c * 16, 16)] for c in range(8)]
        bm1v = [cb[2, pl.ds(c * 16, 16)] for c in range(8)]
        mnv = [cb[3, pl.ds(c * 16, 16)] for c in range(8)]

        def _one_iter(s_in, s_out, cs_in, cs_out, it, do_sync):
            pltpu.sync_copy(cs_in, csb)

            def _zr(r, _):
                for c in range(8):
                    zbuf[r, pl.ds(c * 16, 16)] = jnp.zeros((16,), f32)
                return 0
            lax.fori_loop(0, _RB, _zr, 0)
            for k in range(kblocks):
                pltpu.sync_copy(zbuf,
                                agg_sh.at[pl.ds(sid * rpt + k * _RB, _RB)])
            plsc.subcore_barrier()

            def _stage(j):
                par = (j >> 5) & 1
                boff = pl.multiple_of(tb0 + j, 32)
                pltpu.sync_copy(colg_hbm.at[pl.ds(boff, 32)], cg_vm.at[par])
                pltpu.sync_copy(rowl_hbm.at[pl.ds(boff, 32)], rl_vm.at[par])

            def _gather(j, k):
                par = (j >> 5) & 1
                c = j & 31
                return pltpu.async_copy(s_in.at[cg_vm.at[par].at[c]],
                                        rows.at[k], gsem.at[k])

            def _gwait(j, k):
                par = (j >> 5) & 1
                c = j & 31
                pltpu.make_async_copy(s_in.at[cg_vm.at[par].at[c]],
                                      rows.at[k], gsem.at[k]).wait()

            def _scatter(j, k):
                par = (j >> 5) & 1
                c = j & 31
                return pltpu.async_copy(rows.at[k],
                                        agg_sh.at[rl_vm.at[par].at[c]],
                                        ssem.at[k], add=True)

            def _swait(j, k):
                par = (j >> 5) & 1
                c = j & 31
                pltpu.make_async_copy(rows.at[k],
                                      agg_sh.at[rl_vm.at[par].at[c]],
                                      ssem.at[k]).wait()

            @pl.when(ngrp > 0)
            def _():
                _stage(0)
                for k in range(2):
                    _gather(k, k)

            def _grp(grp, _):
                j0 = 2 * grp
                for k in range(2):
                    _gwait(j0 + k, k)
                    _scatter(j0 + k, k)
                jn0 = j0 + 2

                @pl.when(jn0 < nb)
                def _():
                    @pl.when((jn0 & 31) == 0)
                    def _():
                        _stage(jn0)
                    for k in range(2):
                        _swait(j0 + k, k)
                        _gather(jn0 + k, k)

                @pl.when(jn0 >= nb)
                def _():
                    for k in range(2):
                        _swait(j0 + k, k)
                return 0

            lax.fori_loop(0, ngrp, _grp, 0)
            plsc.subcore_barrier()

            amuv = []
            for c in range(8):
                sl = pl.ds(c * 16, 16)
                mu = (csb[0, sl] + csb[1, sl]) * inv_n
                amuv.append(amv[c] * mu)

            def _bblk(k, accs):
                rb0 = sid * rpt + k * _RB
                grow0 = cid * hp + rb0
                pltpu.sync_copy(agg_sh.at[pl.ds(rb0, _RB)], ablk)
                pltpu.sync_copy(kb_hbm.at[pl.ds(grow0, _RB)], kblk)

                def _rb(r, accs):
                    lrow = rb0 + r
                    dis_r = dis_vm[pl.ds(k * _RB + r, 16)][0]
                    g_r = g_vm[pl.ds(k * _RB + r, 16)][0]
                    val = jnp.where(lrow < realh, 1.0, 0.0).astype(f32)
                    new = []
                    for c in range(8):
                        sl = pl.ds(c * 16, 16)
                        a = ablk[r, sl]
                        v = a * dis_r * alv[c] + amuv[c]
                        kv = kblk[r, sl]
                        bb = bm1v[c] * g_r + 1.0
                        outv = (v * bb + kv * g_r) * val
                        oblk[r, sl] = outv + mnv[c]
                        sblk[r, sl] = outv * dis_r
                        new.append(accs[c] + outv)
                    return tuple(new)

                accs = lax.fori_loop(0, _RB, _rb, accs)
                pltpu.sync_copy(oblk, outp_hbm.at[pl.ds(grow0, _RB)])
                pltpu.sync_copy(sblk, s_out.at[pl.ds(grow0, _RB)])
                return accs

            accs = tuple(jnp.zeros((16,), f32) for _ in range(8))
            accs = lax.fori_loop(0, kblocks, _bblk, accs)
            for c in range(8):
                ablk[0, pl.ds(c * 16, 16)] = accs[c]
            pltpu.sync_copy(ablk.at[pl.ds(0, 1)], part_sh.at[pl.ds(sid, 1)])
            plsc.subcore_barrier()

            @pl.when(sid == 0)
            def _():
                pltpu.sync_copy(part_sh, pbuf)
                for c in range(8):
                    sl = pl.ds(c * 16, 16)
                    tot = pbuf[0, sl]
                    for t in range(1, _NTILES):
                        tot = tot + pbuf[t, sl]
                    ablk[1, sl] = tot
                pltpu.sync_copy(ablk.at[pl.ds(1, 1)],
                                cs_out.at[pl.ds(cid, 1)])

                def _rz(r, _):
                    for c in range(8):
                        zbuf[r, pl.ds(c * 16, 16)] = jnp.zeros((16,), f32)
                    return 0
                lax.fori_loop(0, 16, _rz, 0)
                pltpu.sync_copy(zbuf.at[pl.ds(0, 16)],
                                s_out.at[pl.ds(np_ + cid * 16, 16)])
            plsc.subcore_barrier()
            if do_sync:
                pltpu.core_barrier(bsem, core_axis_name="c")

        def _pair(ph, _):
            it = 2 * ph
            _one_iter(sa_hbm, sb_hbm, cs_hbm, csb_hbm, it, True)
            _one_iter(sb_hbm, sa_hbm, csb_hbm, csa_hbm, it + 1, True)
            return 0

        lax.fori_loop(0, nk >> 1, _pair, 0)

        @pl.when((nk & 1) == 1)
        def _():
            _one_iter(sa_hbm, sb_hbm, cs_hbm, csb_hbm, nk - 1, False)

    return step


def _setup(x, edge_index, know_mask, eta, theta):
    f32 = jnp.float32
    n, d = x.shape
    e = edge_index.shape[1]
    realh = n // 2
    hp = -(-realh // (_NTILES * _RB)) * (_NTILES * _RB)
    np_ = 2 * hp

    row = edge_index[0]
    col = edge_index[1]
    ew = (row != col)
    deg = jax.ops.segment_sum(ew.astype(f32), row, num_segments=n)
    dis = jnp.where(deg > 0, lax.rsqrt(jnp.where(deg > 0, deg, 1.0)), 0.0)

    alpha = (n - 1.0) / (theta * n + n - 1.0)
    beta = 1.0 / alpha / (1.0 / alpha + eta)

    nkm = know_mask.shape[0]
    xk = x[know_mask]
    mean = xk.mean(axis=0)
    upd = jnp.concatenate(
        [xk - mean, jnp.ones((nkm, 1), f32), jnp.zeros((nkm, 15), f32)], 1)
    kext = jnp.zeros((n, d + 16), f32).at[know_mask].set(upd)
    kdense = kext[:, :d]
    gflag = kext[:, d]
    kb = kdense * (1.0 - beta)
    out0 = kdense - mean * (1.0 - gflag)[:, None]

    zpad = hp - realh

    def padrows(a):
        top, bot = a[:realh], a[realh:]
        z = jnp.zeros((zpad,) + a.shape[1:], a.dtype)
        return jnp.concatenate([top, z, bot, z], 0)

    kb_p = padrows(kb)
    dis_p = padrows(dis)
    g_p = padrows(gflag)
    out0_p = padrows(out0)
    s0 = jnp.concatenate([dis_p[:, None] * out0_p, jnp.zeros((_NZ, d), f32)], 0)
    cs0 = jnp.stack([out0[:realh].sum(0), out0[realh:].sum(0)])
    cons = jnp.stack([alpha, 1.0 - alpha, beta - 1.0, mean], 0)

    side = (row >= realh).astype(jnp.int32)
    order = jnp.argsort(side, stable=True)
    row_s = row[order]
    col_s = col[order]
    es = e - side.sum()
    pad0 = (-es) % (32 * _BLK)
    p0 = es + pad0
    cnt1 = e - es
    pblk = -(-(e + 64 * _BLK) // _BLK) + 32
    ptot = pblk * _BLK
    t = jnp.arange(ptot, dtype=jnp.int32)
    src = jnp.where(t < es, t,
                    jnp.where(t < p0, -1,
                              jnp.where(t < p0 + cnt1, t - pad0, -1)))
    valid = src >= 0
    srcc = jnp.clip(src, 0, e - 1)
    rs = row_s[srcc]
    cgs = col_s[srcc]
    colm = cgs + zpad * (cgs >= realh).astype(jnp.int32)
    colg = jnp.where(valid & (rs != cgs), colm, np_ + (t % _NZ))
    colg = colg.astype(jnp.int32).reshape(pblk, _BLK)
    rowl = jnp.where(valid, rs - realh * (rs >= realh).astype(jnp.int32),
                     realh).astype(jnp.int32).reshape(pblk, _BLK)
    b0 = p0 // _BLK
    b1 = 32 * (-(-cnt1 // (32 * _BLK)))

    return dict(n=n, d=d, realh=realh, hp=hp, np_=np_,
                s0=s0, out0_p=out0_p, cs0=cs0, cons=cons, kb_p=kb_p,
                dis_p=dis_p, g_p=g_p, colg=colg, rowl=rowl, b0=b0, b1=b1,
                mean=mean)


def kernel(x, y, edge_index, know_mask, eta, theta, num_iter):
    del y
    st = _setup(x, edge_index, know_mask, eta, theta)
    n, d, realh, hp = st["n"], st["d"], st["realh"], st["hp"]
    step = _build_step(n, d, realh, hp, st["np_"])
    ni = jnp.asarray(num_iter, jnp.int32)
    sizes = (jnp.zeros((16,), jnp.int32)
             .at[0].set(st["b0"]).at[1].set(st["b1"]).at[2].set(ni))
    outp, _, _, _ = step(st["s0"], st["cs0"], st["cons"], st["kb_p"],
                            st["dis_p"], st["g_p"], st["colg"], st["rowl"],
                            sizes)
    res = jnp.concatenate([outp[:realh], outp[hp:hp + realh]], 0)
    init = st["out0_p"] + st["mean"]
    init = jnp.concatenate([init[:realh], init[hp:hp + realh]], 0)
    return jnp.where(ni > 0, res, init)
